# baseline (speedup 1.0000x reference)
import functools

import jax
import jax.numpy as jnp
from jax import lax
from jax.experimental import pallas as pl
from jax.experimental.pallas import tpu as pltpu
from jax.experimental.pallas import tpu_sc as plsc

N_NODES = 10000
N_EDGES = 320000
D_FEAT = 128
D_HALF = D_FEAT // 2

N_TILES = 16
NP = 10240
NPT = NP // N_TILES
EPT = N_EDGES // N_TILES
K = 80
NCHUNKS = EPT // K
NBUF = 5
NMAIN = (NCHUNKS // NBUF) * NBUF
NOUTER = NMAIN // NBUF

_mesh = plsc.VectorSubcoreMesh(core_axis_name="c", subcore_axis_name="s")


def _mainloop(c, eh2, srcall, dstall, acc, cnt, rows, ones_k, semg, sems,
              semc):
    def xform_row(ci):
        for j2 in range(K // 16):
            v = srcall[ci, pl.ds(j2 * 16, 16)]
            srcall[ci, pl.ds(j2 * 16, 16)] = v * 2 + c

    def gather(ci, b):
        return pltpu.async_copy(eh2.at[srcall.at[ci]], rows[b], semg[b])

    def gather_wait(ci, b):
        pltpu.make_async_copy(eh2.at[srcall.at[ci]], rows[b], semg[b]).wait()

    for j in range(NBUF):
        xform_row(j)
        gather(j, j)

    def outer(o, carry):
        for j in range(NBUF):
            ci = o * NBUF + j
            gather_wait(ci, j)
            sd = pltpu.async_copy(rows[j], acc.at[dstall.at[ci]], sems[j],
                                  add=True)

            @pl.when(o > 0)
            def _():
                pltpu.make_async_copy(ones_k, cnt.at[dstall.at[ci]],
                                      semc[j]).wait()

            pltpu.async_copy(ones_k, cnt.at[dstall.at[ci]], semc[j], add=True)

            @pl.when(o < NOUTER - 1)
            def _():
                xform_row(ci + NBUF)

            sd.wait()

            @pl.when(o < NOUTER - 1)
            def _():
                gather(ci + NBUF, j)

        return carry

    lax.fori_loop(0, NOUTER, outer, 0)

    for ci in range(NMAIN, NCHUNKS):
        b = ci % NBUF
        xform_row(ci)
        pltpu.async_copy(eh2.at[srcall.at[ci]], rows[b], semg[b]).wait()
        pltpu.async_copy(rows[b], acc.at[dstall.at[ci]], sems[b],
                         add=True).wait()
        pltpu.async_copy(ones_k, cnt.at[dstall.at[ci]], semc[b],
                         add=True).wait()

    for j in range(NBUF):
        pltpu.make_async_copy(ones_k, cnt.at[dstall.at[0]], semc[j]).wait()


def _ep_round(nrows, rowoff, base, coloff, acc, outbuf, scalebuf, out):
    pltpu.sync_copy(acc.at[pl.ds(base + rowoff, nrows)],
                    outbuf.at[pl.ds(0, nrows)])

    def grp(g, carry):
        sc16 = scalebuf[pl.ds(rowoff + g * 16, 16)]
        for l in range(16):
            scv = sc16[l]
            n = g * 16 + l
            for q in range(D_HALF // 16):
                outbuf[n, pl.ds(q * 16, 16)] = (
                    outbuf[n, pl.ds(q * 16, 16)] * scv)
        return carry

    lax.fori_loop(0, nrows // 16, grp, 0)

    pltpu.sync_copy(outbuf.at[pl.ds(0, nrows)],
                    out.at[pl.ds(base + rowoff, nrows), pl.ds(coloff, D_HALF)])


def _epilogue(nrows, base, coloff, acc, cnt, outbuf, cntbuf, scalebuf, out):
    pltpu.sync_copy(cnt.at[pl.ds(base, NPT)], cntbuf)

    def scl(q, carry):
        v = cntbuf[pl.ds(q * 16, 16)]
        sc = jnp.where(v > 0.0, 1.0 / jnp.maximum(v, 1.0), 0.0)
        scalebuf[pl.ds(q * 16, 16)] = sc
        return carry

    lax.fori_loop(0, NPT // 16, scl, 0)

    _ep_round(min(nrows, NPT // 2), 0, base, coloff, acc, outbuf, scalebuf,
              out)
    if nrows > NPT // 2:
        _ep_round(nrows - NPT // 2, NPT // 2, base, coloff, acc, outbuf,
                  scalebuf, out)


@functools.partial(
    pl.kernel,
    out_type=jax.ShapeDtypeStruct((N_NODES, D_FEAT), jnp.float32),
    mesh=_mesh,
    compiler_params=pltpu.CompilerParams(use_tc_tiling_on_sc=False),
    scratch_types=[
        pltpu.VMEM_SHARED((NP, D_HALF), jnp.float32),
        pltpu.VMEM_SHARED((NP,), jnp.float32),
        pltpu.VMEM((NCHUNKS, K), jnp.int32),
        pltpu.VMEM((NCHUNKS, K), jnp.int32),
        [pltpu.VMEM((K, D_HALF), jnp.float32) for _ in range(NBUF)],
        pltpu.VMEM((K,), jnp.float32),
        pltpu.VMEM((NPT // 2, D_HALF), jnp.float32),
        pltpu.VMEM((NPT,), jnp.float32),
        pltpu.VMEM((NPT,), jnp.float32),
        [pltpu.SemaphoreType.DMA for _ in range(NBUF)],
        [pltpu.SemaphoreType.DMA for _ in range(NBUF)],
        [pltpu.SemaphoreType.DMA for _ in range(NBUF)],
    ],
)
def _dgnn_sc(eh2, src4, dst4, out, acc, cnt, srcall, dstall, rows,
             ones_k, outbuf, cntbuf, scalebuf, semg, sems, semc):
    c = lax.axis_index("c")
    s = lax.axis_index("s")
    base = s * NPT

    zv = jnp.zeros((16,), jnp.float32)
    ov = jnp.ones((16,), jnp.float32)

    def zrow(n, carry):
        for q in range(D_HALF // 16):
            outbuf[n, pl.ds(q * 16, 16)] = zv
        return carry

    lax.fori_loop(0, NPT // 2, zrow, 0)

    def zs(i, carry):
        scalebuf[pl.ds(i * 16, 16)] = zv
        return carry

    lax.fori_loop(0, NPT // 16, zs, 0)

    for j in range(K // 16):
        ones_k[pl.ds(j * 16, 16)] = ov

    pltpu.sync_copy(src4.at[s], srcall)
    pltpu.sync_copy(dst4.at[s], dstall)
    pltpu.sync_copy(outbuf, acc.at[pl.ds(base, NPT // 2)])
    pltpu.sync_copy(outbuf, acc.at[pl.ds(base + NPT // 2, NPT // 2)])
    pltpu.sync_copy(scalebuf, cnt.at[pl.ds(base, NPT)])

    plsc.subcore_barrier()

    _mainloop(c, eh2, srcall, dstall, acc, cnt, rows, ones_k, semg, sems,
              semc)

    plsc.subcore_barrier()

    coloff = c * D_HALF

    @pl.when(s < N_TILES - 1)
    def _():
        _epilogue(NPT, base, coloff, acc, cnt, outbuf, cntbuf, scalebuf, out)

    @pl.when(s == N_TILES - 1)
    def _():
        _epilogue(N_NODES - (N_TILES - 1) * NPT, base, coloff, acc, cnt,
                  outbuf, cntbuf, scalebuf, out)


def kernel(entities, relations, edge_index):
    del relations
    eh2 = entities.reshape(2 * N_NODES, D_HALF)
    src4 = edge_index[0].reshape(N_TILES, NCHUNKS, K)
    dst4 = edge_index[2].reshape(N_TILES, NCHUNKS, K)
    return _dgnn_sc(eh2, src4, dst4)


# speedup vs baseline: 1.1197x; 1.1197x over previous
import functools

import jax
import jax.numpy as jnp
from jax import lax
from jax.experimental import pallas as pl
from jax.experimental.pallas import tpu as pltpu
from jax.experimental.pallas import tpu_sc as plsc

N_NODES = 10000
N_EDGES = 320000
D_FEAT = 128
D_HALF = D_FEAT // 2

N_TILES = 16
NP = 10240
NPT = NP // N_TILES
EPT = N_EDGES // N_TILES
K = 80
NCHUNKS = EPT // K
NBUF = 5
NMAIN = (NCHUNKS // NBUF) * NBUF
NOUTER = NMAIN // NBUF

_mesh = plsc.VectorSubcoreMesh(core_axis_name="c", subcore_axis_name="s")


def _mainloop(c, eh2, srcall, dstall, acc, cnt, rows, ones_k, semg, sems,
              semc):
    def xform_row(ci):
        for j2 in range(K // 16):
            v = srcall[ci, pl.ds(j2 * 16, 16)]
            srcall[ci, pl.ds(j2 * 16, 16)] = v * 2 + c

    def gather(ci, b):
        return pltpu.async_copy(eh2.at[srcall.at[ci]], rows[b], semg[b])

    def gather_wait(ci, b):
        pltpu.make_async_copy(eh2.at[srcall.at[ci]], rows[b], semg[b]).wait()

    for j in range(NBUF):
        xform_row(j)
        gather(j, j)

    def outer(o, carry):
        for j in range(NBUF):
            ci = o * NBUF + j
            gather_wait(ci, j)
            sd = pltpu.async_copy(rows[j], acc.at[dstall.at[ci]], sems[j],
                                  add=True)

            @pl.when(o > 0)
            def _():
                pltpu.make_async_copy(ones_k, cnt.at[dstall.at[ci]],
                                      semc[j]).wait()

            pltpu.async_copy(ones_k, cnt.at[dstall.at[ci]], semc[j], add=True)

            @pl.when(o < NOUTER - 1)
            def _():
                xform_row(ci + NBUF)

            sd.wait()

            @pl.when(o < NOUTER - 1)
            def _():
                gather(ci + NBUF, j)

        return carry

    lax.fori_loop(0, NOUTER, outer, 0)

    for ci in range(NMAIN, NCHUNKS):
        b = ci % NBUF
        xform_row(ci)
        pltpu.async_copy(eh2.at[srcall.at[ci]], rows[b], semg[b]).wait()
        pltpu.async_copy(rows[b], acc.at[dstall.at[ci]], sems[b],
                         add=True).wait()
        pltpu.async_copy(ones_k, cnt.at[dstall.at[ci]], semc[b],
                         add=True).wait()

    for j in range(NBUF):
        pltpu.make_async_copy(ones_k, cnt.at[dstall.at[0]], semc[j]).wait()


EPR = 80


def _epilogue(nrounds, base, coloff, acc, cnt, rows, cntbuf, scalebuf, out,
              semg, sems):
    pltpu.sync_copy(cnt.at[pl.ds(base, NPT)], cntbuf)

    def scl(q, carry):
        v = cntbuf[pl.ds(q * 16, 16)]
        sc = jnp.where(v > 0.0, 1.0 / jnp.maximum(v, 1.0), 0.0)
        scalebuf[pl.ds(q * 16, 16)] = sc
        return carry

    lax.fori_loop(0, NPT // 16, scl, 0)

    def load(r, b):
        pltpu.async_copy(acc.at[pl.ds(base + r * EPR, EPR)], rows[b],
                         semg[b])

    def load_wait(b):
        pltpu.make_async_copy(acc.at[pl.ds(base, EPR)], rows[b],
                              semg[b]).wait()

    def store(r, b):
        pltpu.async_copy(
            rows[b],
            out.at[pl.ds(base + r * EPR, EPR), pl.ds(coloff, D_HALF)],
            sems[b])

    def store_wait(b):
        pltpu.make_async_copy(
            rows[b],
            out.at[pl.ds(base, EPR), pl.ds(coloff, D_HALF)],
            sems[b]).wait()

    nprime = min(NBUF - 1, nrounds)
    for r in range(nprime):
        load(r, r % NBUF)

    for r in range(nrounds):
        b = r % NBUF
        load_wait(b)

        def grp(g, carry):
            sc16 = scalebuf[pl.ds(r * EPR + g * 16, 16)]
            for l in range(16):
                scv = sc16[l]
                n = g * 16 + l
                for q in range(D_HALF // 16):
                    rows[b][n, pl.ds(q * 16, 16)] = (
                        rows[b][n, pl.ds(q * 16, 16)] * scv)
            return carry

        lax.fori_loop(0, EPR // 16, grp, 0)
        store(r, b)
        if r + nprime < nrounds:
            if r >= 1:
                store_wait((r - 1) % NBUF)
            load(r + nprime, (r + nprime) % NBUF)

    for r in range(max(0, nrounds - nprime), nrounds):
        store_wait(r % NBUF)


@functools.partial(
    pl.kernel,
    out_type=jax.ShapeDtypeStruct((N_NODES, D_FEAT), jnp.float32),
    mesh=_mesh,
    compiler_params=pltpu.CompilerParams(use_tc_tiling_on_sc=False),
    scratch_types=[
        pltpu.VMEM_SHARED((NP, D_HALF), jnp.float32),
        pltpu.VMEM_SHARED((NP,), jnp.float32),
        pltpu.VMEM((NCHUNKS, K), jnp.int32),
        pltpu.VMEM((NCHUNKS, K), jnp.int32),
        [pltpu.VMEM((K, D_HALF), jnp.float32) for _ in range(NBUF)],
        pltpu.VMEM((K,), jnp.float32),
        pltpu.VMEM((NPT,), jnp.float32),
        pltpu.VMEM((NPT,), jnp.float32),
        [pltpu.SemaphoreType.DMA for _ in range(NBUF)],
        [pltpu.SemaphoreType.DMA for _ in range(NBUF)],
        [pltpu.SemaphoreType.DMA for _ in range(NBUF)],
    ],
)
def _dgnn_sc(eh2, ei4, out, acc, cnt, srcall, dstall, rows,
             ones_k, cntbuf, scalebuf, semg, sems, semc):
    c = lax.axis_index("c")
    s = lax.axis_index("s")
    base = s * NPT

    zv = jnp.zeros((16,), jnp.float32)
    ov = jnp.ones((16,), jnp.float32)

    def zrow(n, carry):
        for q in range(D_HALF // 16):
            rows[0][n, pl.ds(q * 16, 16)] = zv
        return carry

    lax.fori_loop(0, EPR, zrow, 0)

    def zs(i, carry):
        scalebuf[pl.ds(i * 16, 16)] = zv
        return carry

    lax.fori_loop(0, NPT // 16, zs, 0)

    for j in range(K // 16):
        ones_k[pl.ds(j * 16, 16)] = ov

    pend = []
    pend.append(pltpu.async_copy(ei4.at[0, s], srcall, sems[0]))
    pend.append(pltpu.async_copy(ei4.at[2, s], dstall, sems[1]))
    for r in range(NPT // EPR):
        pend.append(pltpu.async_copy(rows[0], acc.at[pl.ds(base + r * EPR,
                                                           EPR)],
                                     semg[r % NBUF]))
    pend.append(pltpu.async_copy(scalebuf, cnt.at[pl.ds(base, NPT)],
                                 sems[2]))
    for d in pend:
        d.wait()

    plsc.subcore_barrier()

    _mainloop(c, eh2, srcall, dstall, acc, cnt, rows, ones_k, semg, sems,
              semc)

    plsc.subcore_barrier()

    coloff = c * D_HALF

    @pl.when(s < N_TILES - 1)
    def _():
        _epilogue(NPT // EPR, base, coloff, acc, cnt, rows, cntbuf, scalebuf,
                  out, semg, sems)

    @pl.when(s == N_TILES - 1)
    def _():
        _epilogue((N_NODES - (N_TILES - 1) * NPT) // EPR, base, coloff, acc,
                  cnt, rows, cntbuf, scalebuf, out, semg, sems)


def kernel(entities, relations, edge_index):
    del relations
    eh2 = entities.reshape(2 * N_NODES, D_HALF)
    ei4 = edge_index.reshape(3, N_TILES, NCHUNKS, K)
    return _dgnn_sc(eh2, ei4)
